# baseline (speedup 1.0000x reference)
import jax
import jax.numpy as jnp
from jax import lax
from jax.experimental import pallas as pl
from jax.experimental.pallas import tpu as pltpu
from jax.experimental.pallas import tpu_sc as plsc

_N = 10000
_E = 320000
_D = 128
_G = 64

_NC = 2
_NS = 16
_NW = _NC * _NS
_EPW = _E // _NW
_CH = 80
_NCHUNK = _EPW // _CH
_WPT = 624

_R = 1000
_NBLK = _N // _R


import functools


@functools.cache
def _sc_mesh():
    return plsc.VectorSubcoreMesh(
        core_axis_name="c", subcore_axis_name="s",
        num_cores=_NC, num_subcores=_NS)


def _agg_body(hp, epack, aggp, acc, eidx, rows, semg, sems, semi):
    c = lax.axis_index("c")
    s = lax.axis_index("s")
    w = c * _NS + s

    def _zb(k, _):
        i = k // 8
        j = (k % 8) * 16
        rows[0, i, pl.ds(j, 16)] = jnp.zeros((16,), jnp.float32)
        return 0
    lax.fori_loop(0, _CH * 8, _zb, 0)

    base = s * _WPT
    for k in range(7):
        pltpu.sync_copy(rows.at[0], acc.at[pl.ds(base + k * _CH, _CH)])
    pltpu.sync_copy(rows.at[0, pl.ds(0, 64)], acc.at[pl.ds(base + 7 * _CH, 64)])

    @pl.when(s == _NS - 1)
    def _():
        pltpu.sync_copy(rows.at[0, pl.ds(0, 16)], acc.at[pl.ds(_NS * _WPT, 16)])
    plsc.subcore_barrier()

    def _idx(i):
        pltpu.async_copy(epack.at[w, i], eidx.at[i % 4], semi)

    def _idx_wait(i):
        pltpu.make_async_copy(epack.at[w, i], eidx.at[i % 4], semi).wait()

    def _gather(i):
        pltpu.async_copy(hp.at[eidx.at[i % 4, 0]], rows.at[i % 2], semg)

    def _gather_wait(i):
        pltpu.make_async_copy(hp.at[eidx.at[i % 4, 0]], rows.at[i % 2],
                              semg).wait()

    def _scat(i):
        pltpu.async_copy(rows.at[i % 2], acc.at[eidx.at[i % 4, 1]], sems,
                         add=True)

    def _scat_wait(i):
        pltpu.make_async_copy(rows.at[i % 2], acc.at[eidx.at[i % 4, 1]],
                              sems).wait()

    _idx(0)
    _idx(1)
    _idx(2)
    _idx_wait(0)
    _gather(0)

    def _chunk(i, _):
        @pl.when(i >= 1)
        def _():
            _scat_wait(i - 1)

        @pl.when(i + 3 < _NCHUNK)
        def _():
            _idx(i + 3)

        @pl.when(i + 1 < _NCHUNK)
        def _():
            _idx_wait(i + 1)
            _gather(i + 1)
        _gather_wait(i)
        _scat(i)
        return 0
    lax.fori_loop(0, _NCHUNK, _chunk, 0)
    _scat_wait(_NCHUNK - 1)
    plsc.subcore_barrier()

    pltpu.sync_copy(acc.at[pl.ds(base, _WPT)], aggp.at[c, pl.ds(base, _WPT)])

    @pl.when(s == _NS - 1)
    def _():
        r0 = _NS * _WPT
        pltpu.sync_copy(acc.at[pl.ds(r0, 16)], aggp.at[c, pl.ds(r0, 16)])


@functools.cache
def _agg_kernel():
    return pl.kernel(
        _agg_body,
        out_type=jax.ShapeDtypeStruct((_NC, _N, _D), jnp.float32),
        mesh=_sc_mesh(),
        scratch_types=[
            pltpu.VMEM_SHARED((_N, _D), jnp.float32),
            pltpu.VMEM((4, 2, _CH), jnp.int32),
            pltpu.VMEM((2, _CH, _D), jnp.float32),
            pltpu.SemaphoreType.DMA,
            pltpu.SemaphoreType.DMA,
            pltpu.SemaphoreType.DMA,
        ],
    )


_EBLK = 2000
_NEB = _E // _EBLK
_HI = 80


def _degdinv_body(dstb_ref, o_ref, deg_scr):
    i = pl.program_id(0)

    @pl.when(i == 0)
    def _():
        deg_scr[...] = jnp.zeros_like(deg_scr)

    bt = dstb_ref[0]
    hi = bt >> 7
    lo = bt & 127
    oh_hi = (hi == lax.broadcasted_iota(jnp.int32, (_HI, _EBLK), 0)
             ).astype(jnp.bfloat16)
    oh_lo = (lo == lax.broadcasted_iota(jnp.int32, (_D, _EBLK), 0)
             ).astype(jnp.bfloat16)
    deg_scr[...] += lax.dot_general(oh_hi, oh_lo, (((1,), (1,)), ((), ())),
                                    preferred_element_type=jnp.float32)

    @pl.when(i == _NEB - 1)
    def _():
        d = deg_scr[...] + 1.0
        dm = lax.rsqrt(d)
        dm = dm * (1.5 - 0.5 * d * dm * dm)
        ones_b = jnp.ones((_HI, 1, _D), jnp.float32)
        big = lax.dot_general(dm.reshape(_HI, 1, _D), ones_b,
                              (((1,), (1,)), ((0,), (0,))),
                              precision=lax.Precision.HIGHEST,
                              preferred_element_type=jnp.float32)
        o_ref[...] = big.reshape(_HI * _D, _D)


def _dinv_tc(dstb):
    return pl.pallas_call(
        _degdinv_body,
        grid=(_NEB,),
        in_specs=[pl.BlockSpec((1, 1, _EBLK), lambda i: (i, 0, 0))],
        out_specs=pl.BlockSpec((_HI * _D, _D), lambda i: (0, 0)),
        out_shape=jax.ShapeDtypeStruct((_HI * _D, _D), jnp.float32),
        scratch_shapes=[pltpu.VMEM((_HI, _D), jnp.float32)],
    )(dstb)


def _mm_scale_body(x_ref, w_ref, dinv_ref, o_ref):
    h = jnp.dot(x_ref[...], w_ref[...], preferred_element_type=jnp.float32)
    o_ref[...] = h * dinv_ref[...]


def _mm_scale(x, w, dinvf):
    return pl.pallas_call(
        _mm_scale_body,
        grid=(_NBLK,),
        in_specs=[
            pl.BlockSpec((_R, _D), lambda i: (i, 0)),
            pl.BlockSpec((_D, _D), lambda i: (0, 0)),
            pl.BlockSpec((_R, _D), lambda i: (i, 0)),
        ],
        out_specs=pl.BlockSpec((_R, _D), lambda i: (i, 0)),
        out_shape=jax.ShapeDtypeStruct((_N, _D), jnp.float32),
    )(x, w, dinvf)


_BN_S = 0.9999950000374997


def _post_mm_body(agg_ref, hp_ref, dinv_ref, b_ref, g_ref, be_ref, w_ref, o_ref):
    dinv = dinv_ref[...]
    conv = dinv * (agg_ref[0] + agg_ref[1] + hp_ref[...]) + b_ref[...]
    y = jnp.maximum(conv * (g_ref[...] * _BN_S) + be_ref[...], 0.0)
    o_ref[...] = jnp.dot(y, w_ref[...], preferred_element_type=jnp.float32) * dinv


def _post_mm(agg, hp, dinvf, b, g, be, w):
    return pl.pallas_call(
        _post_mm_body,
        grid=(_NBLK,),
        in_specs=[
            pl.BlockSpec((_NC, _R, _D), lambda i: (0, i, 0)),
            pl.BlockSpec((_R, _D), lambda i: (i, 0)),
            pl.BlockSpec((_R, _D), lambda i: (i, 0)),
            pl.BlockSpec((1, _D), lambda i: (0, 0)),
            pl.BlockSpec((1, _D), lambda i: (0, 0)),
            pl.BlockSpec((1, _D), lambda i: (0, 0)),
            pl.BlockSpec((_D, _D), lambda i: (0, 0)),
        ],
        out_specs=pl.BlockSpec((_R, _D), lambda i: (i, 0)),
        out_shape=jax.ShapeDtypeStruct((_N, _D), jnp.float32),
    )(agg, hp, dinvf, b, g, be, w)


def _post_pool_body(agg_ref, hp_ref, dinv_ref, b_ref, g_ref, be_ref, bt_ref,
                    sums_ref, cnt_ref):
    dinv = dinv_ref[...]
    conv = dinv * (agg_ref[0] + agg_ref[1] + hp_ref[...]) + b_ref[...]
    y = jnp.maximum(conv * (g_ref[...] * _BN_S) + be_ref[...], 0.0)
    bt = bt_ref[0]
    seg = lax.broadcasted_iota(jnp.int32, (_G, _R), 0)
    oh = (bt == seg).astype(jnp.float32)
    sums_c = jnp.dot(oh, y, preferred_element_type=jnp.float32)
    cnt_c = jnp.dot(oh, jnp.ones_like(y), preferred_element_type=jnp.float32)

    @pl.when(pl.program_id(0) == 0)
    def _():
        sums_ref[...] = jnp.zeros_like(sums_ref)
        cnt_ref[...] = jnp.zeros_like(cnt_ref)

    sums_ref[...] += sums_c
    cnt_ref[...] += cnt_c


def _post_pool(agg, hp, dinvf, b, g, be, batchr):
    return pl.pallas_call(
        _post_pool_body,
        grid=(_NBLK,),
        in_specs=[
            pl.BlockSpec((_NC, _R, _D), lambda i: (0, i, 0)),
            pl.BlockSpec((_R, _D), lambda i: (i, 0)),
            pl.BlockSpec((_R, _D), lambda i: (i, 0)),
            pl.BlockSpec((1, _D), lambda i: (0, 0)),
            pl.BlockSpec((1, _D), lambda i: (0, 0)),
            pl.BlockSpec((1, _D), lambda i: (0, 0)),
            pl.BlockSpec((1, 1, _R), lambda i: (i, 0, 0)),
        ],
        out_specs=[
            pl.BlockSpec((_G, _D), lambda i: (0, 0)),
            pl.BlockSpec((_G, _D), lambda i: (0, 0)),
        ],
        out_shape=[
            jax.ShapeDtypeStruct((_G, _D), jnp.float32),
            jax.ShapeDtypeStruct((_G, _D), jnp.float32),
        ],
    )(agg, hp, dinvf, b, g, be, batchr)


def _mlp_body(sums_ref, cnt_ref, m1w, m1b, m2w, m2b, m3w, m3b, m4w, m4b, o_ref):
    pooled = sums_ref[...] / jnp.maximum(cnt_ref[...], 1.0)
    z = jnp.maximum(jnp.dot(pooled, m1w[...], preferred_element_type=jnp.float32)
                    + m1b[...], 0.0)
    z = jnp.maximum(jnp.dot(z, m2w[...], preferred_element_type=jnp.float32)
                    + m2b[...], 0.0)
    z = jnp.maximum(jnp.dot(z, m3w[...], preferred_element_type=jnp.float32)
                    + m3b[...], 0.0)
    o_ref[...] = jnp.dot(z, m4w[...], preferred_element_type=jnp.float32) + m4b[...]


def _mlp(sums, cnt, m1w, m1b, m2w, m2b, m3w, m3b, m4w, m4b):
    full = pl.BlockSpec((_D, _D), lambda: (0, 0))
    vec = pl.BlockSpec((1, _D), lambda: (0, 0))
    gd = pl.BlockSpec((_G, _D), lambda: (0, 0))
    return pl.pallas_call(
        _mlp_body,
        in_specs=[gd, gd, full, vec, full, vec, full, vec, full, vec],
        out_specs=gd,
        out_shape=jax.ShapeDtypeStruct((_G, _D), jnp.float32),
    )(sums, cnt, m1w, m1b, m2w, m2b, m3w, m3b, m4w, m4b)


def _pad2(m, rows, cols):
    return jnp.pad(m, ((0, rows - m.shape[0]), (0, cols - m.shape[1])))


def kernel(x, edge_index, batch, W1, b1, g1, be1, W2, b2, g2, be2,
           W3, b3, g3, be3, M1w, M1b, M2w, M2b, M3w, M3b, M4w, M4b):
    srcr = edge_index[0].reshape(_NW, _NCHUNK, _CH)
    dstr = edge_index[1].reshape(_NW, _NCHUNK, _CH)
    epack = jnp.stack([srcr, dstr], axis=2)
    batchr = batch.reshape(_NBLK, 1, _R)

    b1r, g1r, be1r = b1.reshape(1, _D), g1.reshape(1, _D), be1.reshape(1, _D)
    b2r, g2r, be2r = b2.reshape(1, _D), g2.reshape(1, _D), be2.reshape(1, _D)
    b3r, g3r, be3r = b3.reshape(1, _D), g3.reshape(1, _D), be3.reshape(1, _D)
    m1b = M1b.reshape(1, _D)
    m2w, m2b = _pad2(M2w, _D, _D), _pad2(M2b.reshape(1, -1), 1, _D)
    m3w, m3b = _pad2(M3w, _D, _D), _pad2(M3b.reshape(1, -1), 1, _D)
    m4w, m4b = _pad2(M4w, _D, _D), _pad2(M4b.reshape(1, -1), 1, _D)

    dstb = edge_index[1].reshape(_NEB, 1, _EBLK)
    dinvf = _dinv_tc(dstb)
    h1p = _mm_scale(x, W1, dinvf)
    agg1 = _agg_kernel()(h1p, epack)
    h2p = _post_mm(agg1, h1p, dinvf, b1r, g1r, be1r, W2)
    agg2 = _agg_kernel()(h2p, epack)
    h3p = _post_mm(agg2, h2p, dinvf, b2r, g2r, be2r, W3)
    agg3 = _agg_kernel()(h3p, epack)
    sums, cnt = _post_pool(agg3, h3p, dinvf, b3r, g3r, be3r, batchr)
    out = _mlp(sums, cnt, M1w, m1b, m2w, m2b, m3w, m3b, m4w, m4b)
    return out[:, :2]


# speedup vs baseline: 1.0833x; 1.0833x over previous
import jax
import jax.numpy as jnp
from jax import lax
from jax.experimental import pallas as pl
from jax.experimental.pallas import tpu as pltpu
from jax.experimental.pallas import tpu_sc as plsc

_N = 10000
_E = 320000
_D = 128
_G = 64

_NC = 2
_NS = 16
_NW = _NC * _NS
_EPW = _E // _NW
_NP = 10240
_CH = 128
_EPWP = 10112
_NCHUNK = _EPWP // _CH
_WPT = _NP // _NS

_R = 1024
_NBLK = _NP // _R


import functools


@functools.cache
def _sc_mesh():
    return plsc.VectorSubcoreMesh(
        core_axis_name="c", subcore_axis_name="s",
        num_cores=_NC, num_subcores=_NS)


def _agg_body(hp, epack, aggp, acc, eidx, rows, semg, sems, semi):
    c = lax.axis_index("c")
    s = lax.axis_index("s")
    w = c * _NS + s

    def _zb(k, _):
        i = k // 8
        j = (k % 8) * 16
        rows[0, i, pl.ds(j, 16)] = jnp.zeros((16,), jnp.float32)
        return 0
    lax.fori_loop(0, _CH * 8, _zb, 0)

    base = s * _WPT
    for k in range(5):
        pltpu.sync_copy(rows.at[0], acc.at[pl.ds(base + k * _CH, _CH)])
    plsc.subcore_barrier()

    def _idx(i):
        pltpu.async_copy(epack.at[w, i], eidx.at[i % 4], semi)

    def _idx_wait(i):
        pltpu.make_async_copy(epack.at[w, i], eidx.at[i % 4], semi).wait()

    def _gather(i):
        pltpu.async_copy(hp.at[eidx.at[i % 4, 0]], rows.at[i % 2], semg)

    def _gather_wait(i):
        pltpu.make_async_copy(hp.at[eidx.at[i % 4, 0]], rows.at[i % 2],
                              semg).wait()

    def _scat(i):
        pltpu.async_copy(rows.at[i % 2], acc.at[eidx.at[i % 4, 1]], sems,
                         add=True)

    def _scat_wait(i):
        pltpu.make_async_copy(rows.at[i % 2], acc.at[eidx.at[i % 4, 1]],
                              sems).wait()

    _idx(0)
    _idx(1)
    _idx(2)
    _idx_wait(0)
    _gather(0)

    def _chunk(i, _):
        @pl.when(i >= 1)
        def _():
            _scat_wait(i - 1)

        @pl.when(i + 3 < _NCHUNK)
        def _():
            _idx(i + 3)

        @pl.when(i + 1 < _NCHUNK)
        def _():
            _idx_wait(i + 1)
            _gather(i + 1)
        _gather_wait(i)
        _scat(i)
        return 0
    lax.fori_loop(0, _NCHUNK, _chunk, 0)
    _scat_wait(_NCHUNK - 1)
    plsc.subcore_barrier()

    pltpu.sync_copy(acc.at[pl.ds(base, _WPT)], aggp.at[c, pl.ds(base, _WPT)])


@functools.cache
def _agg_kernel():
    return pl.kernel(
        _agg_body,
        out_type=jax.ShapeDtypeStruct((_NC, _NP, _D), jnp.float32),
        mesh=_sc_mesh(),
        scratch_types=[
            pltpu.VMEM_SHARED((_NP, _D), jnp.float32),
            pltpu.VMEM((4, 2, _CH), jnp.int32),
            pltpu.VMEM((2, _CH, _D), jnp.float32),
            pltpu.SemaphoreType.DMA,
            pltpu.SemaphoreType.DMA,
            pltpu.SemaphoreType.DMA,
        ],
    )


_EBLK = 2000
_NEB = _E // _EBLK
_HI = 80


def _degdinv_body(dstb_ref, o_ref, deg_scr):
    i = pl.program_id(0)

    @pl.when(i == 0)
    def _():
        deg_scr[...] = jnp.zeros_like(deg_scr)

    bt = dstb_ref[0]
    hi = bt >> 7
    lo = bt & 127
    oh_hi = (hi == lax.broadcasted_iota(jnp.int32, (_HI, _EBLK), 0)
             ).astype(jnp.bfloat16)
    oh_lo = (lo == lax.broadcasted_iota(jnp.int32, (_D, _EBLK), 0)
             ).astype(jnp.bfloat16)
    deg_scr[...] += lax.dot_general(oh_hi, oh_lo, (((1,), (1,)), ((), ())),
                                    preferred_element_type=jnp.float32)

    @pl.when(i == _NEB - 1)
    def _():
        d = deg_scr[...] + 1.0
        dm = lax.rsqrt(d)
        dm = dm * (1.5 - 0.5 * d * dm * dm)
        ones_b = jnp.ones((_HI, 1, _D), jnp.float32)
        big = lax.dot_general(dm.reshape(_HI, 1, _D), ones_b,
                              (((1,), (1,)), ((0,), (0,))),
                              precision=lax.Precision.HIGHEST,
                              preferred_element_type=jnp.float32)
        o_ref[...] = big.reshape(_HI * _D, _D)


def _dinv_tc(dstb):
    return pl.pallas_call(
        _degdinv_body,
        grid=(_NEB,),
        in_specs=[pl.BlockSpec((1, 1, _EBLK), lambda i: (i, 0, 0))],
        out_specs=pl.BlockSpec((_HI * _D, _D), lambda i: (0, 0)),
        out_shape=jax.ShapeDtypeStruct((_HI * _D, _D), jnp.float32),
        scratch_shapes=[pltpu.VMEM((_HI, _D), jnp.float32)],
    )(dstb)


def _row_valid():
    rows = (lax.broadcasted_iota(jnp.int32, (_R, 1), 0)
            + pl.program_id(0) * _R)
    return rows < _N


def _mm_scale_body(x_ref, w_ref, dinv_ref, o_ref):
    h = jnp.dot(x_ref[...], w_ref[...], preferred_element_type=jnp.float32)
    o_ref[...] = jnp.where(_row_valid(), h * dinv_ref[...], 0.0)


def _mm_scale(x, w, dinvf):
    return pl.pallas_call(
        _mm_scale_body,
        grid=(_NBLK,),
        in_specs=[
            pl.BlockSpec((_R, _D), lambda i: (i, 0)),
            pl.BlockSpec((_D, _D), lambda i: (0, 0)),
            pl.BlockSpec((_R, _D), lambda i: (i, 0)),
        ],
        out_specs=pl.BlockSpec((_R, _D), lambda i: (i, 0)),
        out_shape=jax.ShapeDtypeStruct((_NP, _D), jnp.float32),
    )(x, w, dinvf)


_BN_S = 0.9999950000374997


def _post_mm_body(agg_ref, hp_ref, dinv_ref, b_ref, g_ref, be_ref, w_ref, o_ref):
    dinv = dinv_ref[...]
    conv = dinv * (agg_ref[0] + agg_ref[1] + hp_ref[...]) + b_ref[...]
    y = jnp.maximum(conv * (g_ref[...] * _BN_S) + be_ref[...], 0.0)
    o = jnp.dot(y, w_ref[...], preferred_element_type=jnp.float32) * dinv
    o_ref[...] = jnp.where(_row_valid(), o, 0.0)


def _post_mm(agg, hp, dinvf, b, g, be, w):
    return pl.pallas_call(
        _post_mm_body,
        grid=(_NBLK,),
        in_specs=[
            pl.BlockSpec((_NC, _R, _D), lambda i: (0, i, 0)),
            pl.BlockSpec((_R, _D), lambda i: (i, 0)),
            pl.BlockSpec((_R, _D), lambda i: (i, 0)),
            pl.BlockSpec((1, _D), lambda i: (0, 0)),
            pl.BlockSpec((1, _D), lambda i: (0, 0)),
            pl.BlockSpec((1, _D), lambda i: (0, 0)),
            pl.BlockSpec((_D, _D), lambda i: (0, 0)),
        ],
        out_specs=pl.BlockSpec((_R, _D), lambda i: (i, 0)),
        out_shape=jax.ShapeDtypeStruct((_NP, _D), jnp.float32),
    )(agg, hp, dinvf, b, g, be, w)


def _post_pool_body(agg_ref, hp_ref, dinv_ref, b_ref, g_ref, be_ref, bt_ref,
                    sums_ref, cnt_ref):
    dinv = dinv_ref[...]
    conv = dinv * (agg_ref[0] + agg_ref[1] + hp_ref[...]) + b_ref[...]
    y = jnp.maximum(conv * (g_ref[...] * _BN_S) + be_ref[...], 0.0)
    bt = bt_ref[0]
    seg = lax.broadcasted_iota(jnp.int32, (_G, _R), 0)
    oh = (bt == seg).astype(jnp.float32)
    sums_c = jnp.dot(oh, y, preferred_element_type=jnp.float32)
    cnt_c = jnp.dot(oh, jnp.ones_like(y), preferred_element_type=jnp.float32)

    @pl.when(pl.program_id(0) == 0)
    def _():
        sums_ref[...] = jnp.zeros_like(sums_ref)
        cnt_ref[...] = jnp.zeros_like(cnt_ref)

    sums_ref[...] += sums_c
    cnt_ref[...] += cnt_c


def _post_pool(agg, hp, dinvf, b, g, be, batchr):
    return pl.pallas_call(
        _post_pool_body,
        grid=(_NBLK,),
        in_specs=[
            pl.BlockSpec((_NC, _R, _D), lambda i: (0, i, 0)),
            pl.BlockSpec((_R, _D), lambda i: (i, 0)),
            pl.BlockSpec((_R, _D), lambda i: (i, 0)),
            pl.BlockSpec((1, _D), lambda i: (0, 0)),
            pl.BlockSpec((1, _D), lambda i: (0, 0)),
            pl.BlockSpec((1, _D), lambda i: (0, 0)),
            pl.BlockSpec((1, 1, _R), lambda i: (i, 0, 0)),
        ],
        out_specs=[
            pl.BlockSpec((_G, _D), lambda i: (0, 0)),
            pl.BlockSpec((_G, _D), lambda i: (0, 0)),
        ],
        out_shape=[
            jax.ShapeDtypeStruct((_G, _D), jnp.float32),
            jax.ShapeDtypeStruct((_G, _D), jnp.float32),
        ],
    )(agg, hp, dinvf, b, g, be, batchr)


def _mlp_body(sums_ref, cnt_ref, m1w, m1b, m2w, m2b, m3w, m3b, m4w, m4b, o_ref):
    pooled = sums_ref[...] / jnp.maximum(cnt_ref[...], 1.0)
    z = jnp.maximum(jnp.dot(pooled, m1w[...], preferred_element_type=jnp.float32)
                    + m1b[...], 0.0)
    z = jnp.maximum(jnp.dot(z, m2w[...], preferred_element_type=jnp.float32)
                    + m2b[...], 0.0)
    z = jnp.maximum(jnp.dot(z, m3w[...], preferred_element_type=jnp.float32)
                    + m3b[...], 0.0)
    o_ref[...] = jnp.dot(z, m4w[...], preferred_element_type=jnp.float32) + m4b[...]


def _mlp(sums, cnt, m1w, m1b, m2w, m2b, m3w, m3b, m4w, m4b):
    full = pl.BlockSpec((_D, _D), lambda: (0, 0))
    vec = pl.BlockSpec((1, _D), lambda: (0, 0))
    gd = pl.BlockSpec((_G, _D), lambda: (0, 0))
    return pl.pallas_call(
        _mlp_body,
        in_specs=[gd, gd, full, vec, full, vec, full, vec, full, vec],
        out_specs=gd,
        out_shape=jax.ShapeDtypeStruct((_G, _D), jnp.float32),
    )(sums, cnt, m1w, m1b, m2w, m2b, m3w, m3b, m4w, m4b)


def _pad2(m, rows, cols):
    return jnp.pad(m, ((0, rows - m.shape[0]), (0, cols - m.shape[1])))


def kernel(x, edge_index, batch, W1, b1, g1, be1, W2, b2, g2, be2,
           W3, b3, g3, be3, M1w, M1b, M2w, M2b, M3w, M3b, M4w, M4b):
    pad_rows = _N + (jnp.arange(_NW * (_EPWP - _EPW), dtype=jnp.int32)
                     % (_NP - _N)).reshape(_NW, _EPWP - _EPW)
    srcr = jnp.concatenate([edge_index[0].reshape(_NW, _EPW), pad_rows], 1)
    dstr = jnp.concatenate([edge_index[1].reshape(_NW, _EPW), pad_rows], 1)
    epack = jnp.stack([srcr.reshape(_NW, _NCHUNK, _CH),
                       dstr.reshape(_NW, _NCHUNK, _CH)], axis=2)
    batchr = jnp.pad(batch, (0, _NP - _N), constant_values=_G
                     ).reshape(_NBLK, 1, _R)

    b1r, g1r, be1r = b1.reshape(1, _D), g1.reshape(1, _D), be1.reshape(1, _D)
    b2r, g2r, be2r = b2.reshape(1, _D), g2.reshape(1, _D), be2.reshape(1, _D)
    b3r, g3r, be3r = b3.reshape(1, _D), g3.reshape(1, _D), be3.reshape(1, _D)
    m1b = M1b.reshape(1, _D)
    m2w, m2b = _pad2(M2w, _D, _D), _pad2(M2b.reshape(1, -1), 1, _D)
    m3w, m3b = _pad2(M3w, _D, _D), _pad2(M3b.reshape(1, -1), 1, _D)
    m4w, m4b = _pad2(M4w, _D, _D), _pad2(M4b.reshape(1, -1), 1, _D)

    dstb = edge_index[1].reshape(_NEB, 1, _EBLK)
    dinvf = _dinv_tc(dstb)
    h1p = _mm_scale(x, W1, dinvf)
    agg1 = _agg_kernel()(h1p, epack)
    h2p = _post_mm(agg1, h1p, dinvf, b1r, g1r, be1r, W2)
    agg2 = _agg_kernel()(h2p, epack)
    h3p = _post_mm(agg2, h2p, dinvf, b2r, g2r, be2r, W3)
    agg3 = _agg_kernel()(h3p, epack)
    sums, cnt = _post_pool(agg3, h3p, dinvf, b3r, g3r, be3r, batchr)
    out = _mlp(sums, cnt, M1w, m1b, m2w, m2b, m3w, m3b, m4w, m4b)
    return out[:, :2]


# speedup vs baseline: 1.2040x; 1.1114x over previous
import jax
import jax.numpy as jnp
from jax import lax
from jax.experimental import pallas as pl
from jax.experimental.pallas import tpu as pltpu
from jax.experimental.pallas import tpu_sc as plsc

_N = 10000
_E = 320000
_D = 128
_G = 64

_NC = 2
_NS = 16
_NW = _NC * _NS
_EPW = _E // _NW
_NP = 10240
_CH = 128
_EPWP = 10112
_NCHUNK = _EPWP // _CH
_WPT = _NP // _NS

_R = 1024
_NBLK = _NP // _R


import functools


@functools.cache
def _sc_mesh():
    return plsc.VectorSubcoreMesh(
        core_axis_name="c", subcore_axis_name="s",
        num_cores=_NC, num_subcores=_NS)


def _agg_body(hp, epack, aggp, acc, eidx, rows, semg, sems, semi):
    c = lax.axis_index("c")
    s = lax.axis_index("s")
    w = c * _NS + s

    def _zb(k, _):
        i = k // 8
        j = (k % 8) * 16
        rows[0, i, pl.ds(j, 16)] = jnp.zeros((16,), jnp.float32)
        return 0
    lax.fori_loop(0, _CH * 8, _zb, 0)

    base = s * _WPT
    for k in range(5):
        pltpu.sync_copy(rows.at[0], acc.at[pl.ds(base + k * _CH, _CH)])
    plsc.subcore_barrier()

    def _idx(i):
        pltpu.async_copy(epack.at[w, i], eidx.at[i % 4], semi)

    def _idx_wait(i):
        pltpu.make_async_copy(epack.at[w, i], eidx.at[i % 4], semi).wait()

    def _gather(i):
        pltpu.async_copy(hp.at[eidx.at[i % 4, 0]], rows.at[i % 2], semg)

    def _gather_wait(i):
        pltpu.make_async_copy(hp.at[eidx.at[i % 4, 0]], rows.at[i % 2],
                              semg).wait()

    def _scat(i):
        pltpu.async_copy(rows.at[i % 2], acc.at[eidx.at[i % 4, 1]], sems,
                         add=True)

    def _scat_wait(i):
        pltpu.make_async_copy(rows.at[i % 2], acc.at[eidx.at[i % 4, 1]],
                              sems).wait()

    _idx(0)
    _idx(1)
    _idx(2)
    _idx_wait(0)
    _gather(0)

    def _chunk(i, _):
        @pl.when(i >= 1)
        def _():
            _scat_wait(i - 1)

        @pl.when(i + 3 < _NCHUNK)
        def _():
            _idx(i + 3)

        @pl.when(i + 1 < _NCHUNK)
        def _():
            _idx_wait(i + 1)
            _gather(i + 1)
        _gather_wait(i)
        _scat(i)
        return 0
    lax.fori_loop(0, _NCHUNK, _chunk, 0)
    _scat_wait(_NCHUNK - 1)
    plsc.subcore_barrier()

    pltpu.sync_copy(acc.at[pl.ds(base, _WPT)], aggp.at[c, pl.ds(base, _WPT)])


@functools.cache
def _agg_kernel():
    return pl.kernel(
        _agg_body,
        out_type=jax.ShapeDtypeStruct((_NC, _NP, _D), jnp.float32),
        mesh=_sc_mesh(),
        scratch_types=[
            pltpu.VMEM_SHARED((_NP, _D), jnp.float32),
            pltpu.VMEM((4, 2, _CH), jnp.int32),
            pltpu.VMEM((2, _CH, _D), jnp.float32),
            pltpu.SemaphoreType.DMA,
            pltpu.SemaphoreType.DMA,
            pltpu.SemaphoreType.DMA,
        ],
    )


_EBLK = 8000
_NEB = _E // _EBLK
_HI = 80


def _degdinv_body(dstb_ref, o_ref, deg_scr):
    i = pl.program_id(0)

    @pl.when(i == 0)
    def _():
        deg_scr[...] = jnp.zeros_like(deg_scr)

    bt = dstb_ref[0]
    hi = bt >> 7
    lo = bt & 127
    oh_hi = (hi == lax.broadcasted_iota(jnp.int32, (_HI, _EBLK), 0)
             ).astype(jnp.bfloat16)
    oh_lo = (lo == lax.broadcasted_iota(jnp.int32, (_D, _EBLK), 0)
             ).astype(jnp.bfloat16)
    deg_scr[...] += lax.dot_general(oh_hi, oh_lo, (((1,), (1,)), ((), ())),
                                    preferred_element_type=jnp.float32)

    @pl.when(i == _NEB - 1)
    def _():
        d = deg_scr[...] + 1.0
        dm = lax.rsqrt(d)
        dm = dm * (1.5 - 0.5 * d * dm * dm)
        ones_b = jnp.ones((_HI, 1, _D), jnp.float32)
        big = lax.dot_general(dm.reshape(_HI, 1, _D), ones_b,
                              (((1,), (1,)), ((0,), (0,))),
                              precision=lax.Precision.HIGHEST,
                              preferred_element_type=jnp.float32)
        o_ref[...] = big.reshape(_HI * _D, _D)


def _dinv_tc(dstb):
    return pl.pallas_call(
        _degdinv_body,
        grid=(_NEB,),
        in_specs=[pl.BlockSpec((1, 1, _EBLK), lambda i: (i, 0, 0))],
        out_specs=pl.BlockSpec((_HI * _D, _D), lambda i: (0, 0)),
        out_shape=jax.ShapeDtypeStruct((_HI * _D, _D), jnp.float32),
        scratch_shapes=[pltpu.VMEM((_HI, _D), jnp.float32)],
    )(dstb)


def _row_valid():
    rows = (lax.broadcasted_iota(jnp.int32, (_R, 1), 0)
            + pl.program_id(0) * _R)
    return rows < _N


def _mm_scale_body(x_ref, w_ref, dinv_ref, o_ref):
    h = jnp.dot(x_ref[...], w_ref[...], preferred_element_type=jnp.float32)
    o_ref[...] = jnp.where(_row_valid(), h * dinv_ref[...], 0.0)


def _mm_scale(x, w, dinvf):
    return pl.pallas_call(
        _mm_scale_body,
        grid=(_NBLK,),
        in_specs=[
            pl.BlockSpec((_R, _D), lambda i: (i, 0)),
            pl.BlockSpec((_D, _D), lambda i: (0, 0)),
            pl.BlockSpec((_R, _D), lambda i: (i, 0)),
        ],
        out_specs=pl.BlockSpec((_R, _D), lambda i: (i, 0)),
        out_shape=jax.ShapeDtypeStruct((_NP, _D), jnp.float32),
    )(x, w, dinvf)


_BN_S = 0.9999950000374997


def _post_mm_body(agg_ref, hp_ref, dinv_ref, b_ref, g_ref, be_ref, w_ref, o_ref):
    dinv = dinv_ref[...]
    conv = dinv * (agg_ref[0] + agg_ref[1] + hp_ref[...]) + b_ref[...]
    y = jnp.maximum(conv * (g_ref[...] * _BN_S) + be_ref[...], 0.0)
    o = jnp.dot(y, w_ref[...], preferred_element_type=jnp.float32) * dinv
    o_ref[...] = jnp.where(_row_valid(), o, 0.0)


def _post_mm(agg, hp, dinvf, b, g, be, w):
    return pl.pallas_call(
        _post_mm_body,
        grid=(_NBLK,),
        in_specs=[
            pl.BlockSpec((_NC, _R, _D), lambda i: (0, i, 0)),
            pl.BlockSpec((_R, _D), lambda i: (i, 0)),
            pl.BlockSpec((_R, _D), lambda i: (i, 0)),
            pl.BlockSpec((1, _D), lambda i: (0, 0)),
            pl.BlockSpec((1, _D), lambda i: (0, 0)),
            pl.BlockSpec((1, _D), lambda i: (0, 0)),
            pl.BlockSpec((_D, _D), lambda i: (0, 0)),
        ],
        out_specs=pl.BlockSpec((_R, _D), lambda i: (i, 0)),
        out_shape=jax.ShapeDtypeStruct((_NP, _D), jnp.float32),
    )(agg, hp, dinvf, b, g, be, w)


def _post_pool_body(agg_ref, hp_ref, dinv_ref, b_ref, g_ref, be_ref, bt_ref,
                    m1w, m1b, m2w, m2b, m3w, m3b, m4w, m4b, o_ref,
                    sums_ref, cnt_ref):
    dinv = dinv_ref[...]
    conv = dinv * (agg_ref[0] + agg_ref[1] + hp_ref[...]) + b_ref[...]
    y = jnp.maximum(conv * (g_ref[...] * _BN_S) + be_ref[...], 0.0)
    bt = bt_ref[0]
    seg = lax.broadcasted_iota(jnp.int32, (_G, _R), 0)
    oh = (bt == seg).astype(jnp.float32)
    sums_c = jnp.dot(oh, y, preferred_element_type=jnp.float32)
    cnt_c = jnp.dot(oh, jnp.ones_like(y), preferred_element_type=jnp.float32)

    @pl.when(pl.program_id(0) == 0)
    def _():
        sums_ref[...] = jnp.zeros_like(sums_ref)
        cnt_ref[...] = jnp.zeros_like(cnt_ref)

    sums_ref[...] += sums_c
    cnt_ref[...] += cnt_c

    @pl.when(pl.program_id(0) == _NBLK - 1)
    def _():
        pooled = sums_ref[...] / jnp.maximum(cnt_ref[...], 1.0)
        z = jnp.maximum(jnp.dot(pooled, m1w[...],
                                preferred_element_type=jnp.float32)
                        + m1b[...], 0.0)
        z = jnp.maximum(jnp.dot(z, m2w[...],
                                preferred_element_type=jnp.float32)
                        + m2b[...], 0.0)
        z = jnp.maximum(jnp.dot(z, m3w[...],
                                preferred_element_type=jnp.float32)
                        + m3b[...], 0.0)
        o_ref[...] = (jnp.dot(z, m4w[...],
                              preferred_element_type=jnp.float32)
                      + m4b[...])


def _post_pool_mlp(agg, hp, dinvf, b, g, be, batchr,
                   m1w, m1b, m2w, m2b, m3w, m3b, m4w, m4b):
    full = pl.BlockSpec((_D, _D), lambda i: (0, 0))
    vec = pl.BlockSpec((1, _D), lambda i: (0, 0))
    gd = pl.BlockSpec((_G, _D), lambda i: (0, 0))
    out, _, _ = pl.pallas_call(
        _post_pool_body,
        grid=(_NBLK,),
        in_specs=[
            pl.BlockSpec((_NC, _R, _D), lambda i: (0, i, 0)),
            pl.BlockSpec((_R, _D), lambda i: (i, 0)),
            pl.BlockSpec((_R, _D), lambda i: (i, 0)),
            vec, vec, vec,
            pl.BlockSpec((1, 1, _R), lambda i: (i, 0, 0)),
            full, vec, full, vec, full, vec, full, vec,
        ],
        out_specs=[gd, gd, gd],
        out_shape=[
            jax.ShapeDtypeStruct((_G, _D), jnp.float32),
            jax.ShapeDtypeStruct((_G, _D), jnp.float32),
            jax.ShapeDtypeStruct((_G, _D), jnp.float32),
        ],
    )(agg, hp, dinvf, b, g, be, batchr,
      m1w, m1b, m2w, m2b, m3w, m3b, m4w, m4b)
    return out


def _pad2(m, rows, cols):
    return jnp.pad(m, ((0, rows - m.shape[0]), (0, cols - m.shape[1])))


def kernel(x, edge_index, batch, W1, b1, g1, be1, W2, b2, g2, be2,
           W3, b3, g3, be3, M1w, M1b, M2w, M2b, M3w, M3b, M4w, M4b):
    pad_rows = _N + (jnp.arange(_NW * (_EPWP - _EPW), dtype=jnp.int32)
                     % (_NP - _N)).reshape(_NW, _EPWP - _EPW)
    srcr = jnp.concatenate([edge_index[0].reshape(_NW, _EPW), pad_rows], 1)
    dstr = jnp.concatenate([edge_index[1].reshape(_NW, _EPW), pad_rows], 1)
    epack = jnp.stack([srcr.reshape(_NW, _NCHUNK, _CH),
                       dstr.reshape(_NW, _NCHUNK, _CH)], axis=2)
    batchr = jnp.pad(batch, (0, _NP - _N), constant_values=_G
                     ).reshape(_NBLK, 1, _R)

    b1r, g1r, be1r = b1.reshape(1, _D), g1.reshape(1, _D), be1.reshape(1, _D)
    b2r, g2r, be2r = b2.reshape(1, _D), g2.reshape(1, _D), be2.reshape(1, _D)
    b3r, g3r, be3r = b3.reshape(1, _D), g3.reshape(1, _D), be3.reshape(1, _D)
    m1b = M1b.reshape(1, _D)
    m2w, m2b = _pad2(M2w, _D, _D), _pad2(M2b.reshape(1, -1), 1, _D)
    m3w, m3b = _pad2(M3w, _D, _D), _pad2(M3b.reshape(1, -1), 1, _D)
    m4w, m4b = _pad2(M4w, _D, _D), _pad2(M4b.reshape(1, -1), 1, _D)

    dstb = edge_index[1].reshape(_NEB, 1, _EBLK)
    dinvf = _dinv_tc(dstb)
    h1p = _mm_scale(x, W1, dinvf)
    agg1 = _agg_kernel()(h1p, epack)
    h2p = _post_mm(agg1, h1p, dinvf, b1r, g1r, be1r, W2)
    agg2 = _agg_kernel()(h2p, epack)
    h3p = _post_mm(agg2, h2p, dinvf, b2r, g2r, be2r, W3)
    agg3 = _agg_kernel()(h3p, epack)
    out = _post_pool_mlp(agg3, h3p, dinvf, b3r, g3r, be3r, batchr,
                         M1w, m1b, m2w, m2b, m3w, m3b, m4w, m4b)
    return out[:, :2]


# speedup vs baseline: 1.2138x; 1.0081x over previous
import jax
import jax.numpy as jnp
from jax import lax
from jax.experimental import pallas as pl
from jax.experimental.pallas import tpu as pltpu
from jax.experimental.pallas import tpu_sc as plsc

_N = 10000
_E = 320000
_D = 128
_G = 64

_NC = 2
_NS = 16
_NW = _NC * _NS
_EPW = _E // _NW
_NP = 10240
_CH = 128
_EPWP = 10112
_NCHUNK = _EPWP // _CH
_WPT = _NP // _NS

_R = 1024
_NBLK = _NP // _R


import functools


@functools.cache
def _sc_mesh():
    return plsc.VectorSubcoreMesh(
        core_axis_name="c", subcore_axis_name="s",
        num_cores=_NC, num_subcores=_NS)


def _agg_body(hp, epack, aggp, acc, eidx, rows, semg, sems, semi):
    c = lax.axis_index("c")
    s = lax.axis_index("s")
    w = c * _NS + s

    def _zb(k, _):
        i = k // 8
        j = (k % 8) * 16
        rows[0, i, pl.ds(j, 16)] = jnp.zeros((16,), jnp.float32)
        return 0
    lax.fori_loop(0, _CH * 8, _zb, 0)

    base = s * _WPT
    for k in range(5):
        pltpu.sync_copy(rows.at[0], acc.at[pl.ds(base + k * _CH, _CH)])
    plsc.subcore_barrier()

    def _idx(i):
        pltpu.async_copy(epack.at[w, i], eidx.at[i % 4], semi)

    def _idx_wait(i):
        pltpu.make_async_copy(epack.at[w, i], eidx.at[i % 4], semi).wait()

    def _gather(i):
        pltpu.async_copy(hp.at[eidx.at[i % 4, 0]], rows.at[i % 2], semg)

    def _gather_wait(i):
        pltpu.make_async_copy(hp.at[eidx.at[i % 4, 0]], rows.at[i % 2],
                              semg).wait()

    def _scat(i):
        pltpu.async_copy(rows.at[i % 2], acc.at[eidx.at[i % 4, 1]], sems,
                         add=True)

    def _scat_wait(i):
        pltpu.make_async_copy(rows.at[i % 2], acc.at[eidx.at[i % 4, 1]],
                              sems).wait()

    _idx(0)
    _idx(1)
    _idx(2)
    _idx_wait(0)
    _gather(0)

    def _chunk(i, _):
        @pl.when(i >= 1)
        def _():
            _scat_wait(i - 1)

        @pl.when(i + 3 < _NCHUNK)
        def _():
            _idx(i + 3)

        @pl.when(i + 1 < _NCHUNK)
        def _():
            _idx_wait(i + 1)
            _gather(i + 1)
        _gather_wait(i)
        _scat(i)
        return 0
    lax.fori_loop(0, _NCHUNK, _chunk, 0)
    _scat_wait(_NCHUNK - 1)
    plsc.subcore_barrier()

    pltpu.sync_copy(acc.at[pl.ds(base, _WPT)], aggp.at[c, pl.ds(base, _WPT)])


@functools.cache
def _agg_kernel():
    return pl.kernel(
        _agg_body,
        out_type=jax.ShapeDtypeStruct((_NC, _NP, _D), jnp.float32),
        mesh=_sc_mesh(),
        scratch_types=[
            pltpu.VMEM_SHARED((_NP, _D), jnp.float32),
            pltpu.VMEM((4, 2, _CH), jnp.int32),
            pltpu.VMEM((2, _CH, _D), jnp.float32),
            pltpu.SemaphoreType.DMA,
            pltpu.SemaphoreType.DMA,
            pltpu.SemaphoreType.DMA,
        ],
    )


_EBLK = 8000
_NEB = _E // _EBLK
_HI = 80


def _row_valid():
    rows = (lax.broadcasted_iota(jnp.int32, (_R, 1), 0)
            + pl.program_id(0) * _R)
    return rows < _N


def _mm_scale_body(dstb_ref, x_ref, w_ref, hp_ref, dinv_ref, dinv_scr):
    i = pl.program_id(0)

    @pl.when(i == 0)
    def _():
        def _hist(j, acc):
            bt = dstb_ref[j]
            hi = bt >> 7
            lo = bt & 127
            oh_hi = (hi == lax.broadcasted_iota(jnp.int32, (_HI, _EBLK), 0)
                     ).astype(jnp.bfloat16)
            oh_lo = (lo == lax.broadcasted_iota(jnp.int32, (_D, _EBLK), 0)
                     ).astype(jnp.bfloat16)
            return acc + lax.dot_general(
                oh_hi, oh_lo, (((1,), (1,)), ((), ())),
                preferred_element_type=jnp.float32)
        deg = lax.fori_loop(0, _NEB, _hist, jnp.zeros((_HI, _D), jnp.float32))
        d = deg + 1.0
        dm = lax.rsqrt(d)
        dm = dm * (1.5 - 0.5 * d * dm * dm)
        ones_b = jnp.ones((_HI, 1, _D), jnp.float32)
        big = lax.dot_general(dm.reshape(_HI, 1, _D), ones_b,
                              (((1,), (1,)), ((0,), (0,))),
                              precision=lax.Precision.HIGHEST,
                              preferred_element_type=jnp.float32)
        dinv_scr[...] = big.reshape(_NP, _D)

    dinv = dinv_scr[pl.ds(i * _R, _R), :]
    dinv_ref[...] = dinv
    h = jnp.dot(x_ref[...], w_ref[...], preferred_element_type=jnp.float32)
    hp_ref[...] = jnp.where(_row_valid(), h * dinv, 0.0)


def _mm_scale(x, w, dstb):
    return pl.pallas_call(
        _mm_scale_body,
        grid=(_NBLK,),
        in_specs=[
            pl.BlockSpec((_NEB, 1, _EBLK), lambda i: (0, 0, 0)),
            pl.BlockSpec((_R, _D), lambda i: (i, 0)),
            pl.BlockSpec((_D, _D), lambda i: (0, 0)),
        ],
        out_specs=[
            pl.BlockSpec((_R, _D), lambda i: (i, 0)),
            pl.BlockSpec((_R, _D), lambda i: (i, 0)),
        ],
        out_shape=[
            jax.ShapeDtypeStruct((_NP, _D), jnp.float32),
            jax.ShapeDtypeStruct((_NP, _D), jnp.float32),
        ],
        scratch_shapes=[pltpu.VMEM((_NP, _D), jnp.float32)],
    )(dstb, x, w)


_BN_S = 0.9999950000374997


def _post_mm_body(agg_ref, hp_ref, dinv_ref, b_ref, g_ref, be_ref, w_ref, o_ref):
    dinv = dinv_ref[...]
    conv = dinv * (agg_ref[0] + agg_ref[1] + hp_ref[...]) + b_ref[...]
    y = jnp.maximum(conv * (g_ref[...] * _BN_S) + be_ref[...], 0.0)
    o = jnp.dot(y, w_ref[...], preferred_element_type=jnp.float32) * dinv
    o_ref[...] = jnp.where(_row_valid(), o, 0.0)


def _post_mm(agg, hp, dinvf, b, g, be, w):
    return pl.pallas_call(
        _post_mm_body,
        grid=(_NBLK,),
        in_specs=[
            pl.BlockSpec((_NC, _R, _D), lambda i: (0, i, 0)),
            pl.BlockSpec((_R, _D), lambda i: (i, 0)),
            pl.BlockSpec((_R, _D), lambda i: (i, 0)),
            pl.BlockSpec((1, _D), lambda i: (0, 0)),
            pl.BlockSpec((1, _D), lambda i: (0, 0)),
            pl.BlockSpec((1, _D), lambda i: (0, 0)),
            pl.BlockSpec((_D, _D), lambda i: (0, 0)),
        ],
        out_specs=pl.BlockSpec((_R, _D), lambda i: (i, 0)),
        out_shape=jax.ShapeDtypeStruct((_NP, _D), jnp.float32),
    )(agg, hp, dinvf, b, g, be, w)


def _post_pool_body(agg_ref, hp_ref, dinv_ref, b_ref, g_ref, be_ref, bt_ref,
                    m1w, m1b, m2w, m2b, m3w, m3b, m4w, m4b, o_ref,
                    sums_ref, cnt_ref):
    dinv = dinv_ref[...]
    conv = dinv * (agg_ref[0] + agg_ref[1] + hp_ref[...]) + b_ref[...]
    y = jnp.maximum(conv * (g_ref[...] * _BN_S) + be_ref[...], 0.0)
    bt = bt_ref[0]
    seg = lax.broadcasted_iota(jnp.int32, (_G, _R), 0)
    oh = (bt == seg).astype(jnp.float32)
    sums_c = jnp.dot(oh, y, preferred_element_type=jnp.float32)
    cnt_c = jnp.dot(oh, jnp.ones_like(y), preferred_element_type=jnp.float32)

    @pl.when(pl.program_id(0) == 0)
    def _():
        sums_ref[...] = jnp.zeros_like(sums_ref)
        cnt_ref[...] = jnp.zeros_like(cnt_ref)

    sums_ref[...] += sums_c
    cnt_ref[...] += cnt_c

    @pl.when(pl.program_id(0) == _NBLK - 1)
    def _():
        pooled = sums_ref[...] / jnp.maximum(cnt_ref[...], 1.0)
        z = jnp.maximum(jnp.dot(pooled, m1w[...],
                                preferred_element_type=jnp.float32)
                        + m1b[...], 0.0)
        z = jnp.maximum(jnp.dot(z, m2w[...],
                                preferred_element_type=jnp.float32)
                        + m2b[...], 0.0)
        z = jnp.maximum(jnp.dot(z, m3w[...],
                                preferred_element_type=jnp.float32)
                        + m3b[...], 0.0)
        o_ref[...] = (jnp.dot(z, m4w[...],
                              preferred_element_type=jnp.float32)
                      + m4b[...])


def _post_pool_mlp(agg, hp, dinvf, b, g, be, batchr,
                   m1w, m1b, m2w, m2b, m3w, m3b, m4w, m4b):
    full = pl.BlockSpec((_D, _D), lambda i: (0, 0))
    vec = pl.BlockSpec((1, _D), lambda i: (0, 0))
    gd = pl.BlockSpec((_G, _D), lambda i: (0, 0))
    out, _, _ = pl.pallas_call(
        _post_pool_body,
        grid=(_NBLK,),
        in_specs=[
            pl.BlockSpec((_NC, _R, _D), lambda i: (0, i, 0)),
            pl.BlockSpec((_R, _D), lambda i: (i, 0)),
            pl.BlockSpec((_R, _D), lambda i: (i, 0)),
            vec, vec, vec,
            pl.BlockSpec((1, 1, _R), lambda i: (i, 0, 0)),
            full, vec, full, vec, full, vec, full, vec,
        ],
        out_specs=[gd, gd, gd],
        out_shape=[
            jax.ShapeDtypeStruct((_G, _D), jnp.float32),
            jax.ShapeDtypeStruct((_G, _D), jnp.float32),
            jax.ShapeDtypeStruct((_G, _D), jnp.float32),
        ],
    )(agg, hp, dinvf, b, g, be, batchr,
      m1w, m1b, m2w, m2b, m3w, m3b, m4w, m4b)
    return out


def _pad2(m, rows, cols):
    return jnp.pad(m, ((0, rows - m.shape[0]), (0, cols - m.shape[1])))


def kernel(x, edge_index, batch, W1, b1, g1, be1, W2, b2, g2, be2,
           W3, b3, g3, be3, M1w, M1b, M2w, M2b, M3w, M3b, M4w, M4b):
    pad_rows = _N + (jnp.arange(_NW * (_EPWP - _EPW), dtype=jnp.int32)
                     % (_NP - _N)).reshape(_NW, _EPWP - _EPW)
    srcr = jnp.concatenate([edge_index[0].reshape(_NW, _EPW), pad_rows], 1)
    dstr = jnp.concatenate([edge_index[1].reshape(_NW, _EPW), pad_rows], 1)
    epack = jnp.stack([srcr.reshape(_NW, _NCHUNK, _CH),
                       dstr.reshape(_NW, _NCHUNK, _CH)], axis=2)
    batchr = jnp.pad(batch, (0, _NP - _N), constant_values=_G
                     ).reshape(_NBLK, 1, _R)

    b1r, g1r, be1r = b1.reshape(1, _D), g1.reshape(1, _D), be1.reshape(1, _D)
    b2r, g2r, be2r = b2.reshape(1, _D), g2.reshape(1, _D), be2.reshape(1, _D)
    b3r, g3r, be3r = b3.reshape(1, _D), g3.reshape(1, _D), be3.reshape(1, _D)
    m1b = M1b.reshape(1, _D)
    m2w, m2b = _pad2(M2w, _D, _D), _pad2(M2b.reshape(1, -1), 1, _D)
    m3w, m3b = _pad2(M3w, _D, _D), _pad2(M3b.reshape(1, -1), 1, _D)
    m4w, m4b = _pad2(M4w, _D, _D), _pad2(M4b.reshape(1, -1), 1, _D)

    dstb = edge_index[1].reshape(_NEB, 1, _EBLK)
    h1p, dinvf = _mm_scale(x, W1, dstb)
    agg1 = _agg_kernel()(h1p, epack)
    h2p = _post_mm(agg1, h1p, dinvf, b1r, g1r, be1r, W2)
    agg2 = _agg_kernel()(h2p, epack)
    h3p = _post_mm(agg2, h2p, dinvf, b2r, g2r, be2r, W3)
    agg3 = _agg_kernel()(h3p, epack)
    out = _post_pool_mlp(agg3, h3p, dinvf, b3r, g3r, be3r, batchr,
                         M1w, m1b, m2w, m2b, m3w, m3b, m4w, m4b)
    return out[:, :2]
